# baseline (speedup 1.0000x reference)
import functools

import jax
import jax.numpy as jnp
from jax import lax
from jax.experimental import pallas as pl
from jax.experimental.pallas import tpu as pltpu
from jax.experimental.pallas import tpu_sc as plsc

L = 16
NOBJ = 16
OVERLAP_THRESHOLD = 0.5
VAR0 = 0.1
VAR1 = 0.2
LN2 = 0.6931471805599453
SQRT2 = 1.4142135623730951


def _log_f32(x):
    xi = lax.bitcast_convert_type(x, jnp.int32)
    e = (xi >> 23) - 127
    mi = (xi & 0x007FFFFF) | 0x3F800000
    m = lax.bitcast_convert_type(mi, jnp.float32)
    big = m > SQRT2
    m = jnp.where(big, m * 0.5, m)
    e = e + jnp.where(big, 1, 0)
    s = (m - 1.0) / (m + 1.0)
    s2 = s * s
    p = 1.0 / 9.0 + s2 * 0.0
    p = 1.0 / 7.0 + s2 * p
    p = 1.0 / 5.0 + s2 * p
    p = 1.0 / 3.0 + s2 * p
    p = 1.0 + s2 * p
    return e.astype(jnp.float32) * LN2 + 2.0 * s * p


def _sc_match(anc, tgt, B, Apad):
    NCHUNK = Apad // L
    mesh = plsc.VectorSubcoreMesh(core_axis_name="c", subcore_axis_name="s")

    @functools.partial(
        pl.kernel,
        out_type=(
            jax.ShapeDtypeStruct((B, 4, Apad), jnp.float32),
            jax.ShapeDtypeStruct((B, Apad), jnp.int32),
        ),
        mesh=mesh,
        compiler_params=pltpu.CompilerParams(needs_layout_passes=False),
        scratch_types=[
            pltpu.VMEM((4, Apad), jnp.float32),
            pltpu.VMEM((5, NOBJ), jnp.float32),
            pltpu.VMEM((5, Apad), jnp.float32),
            pltpu.VMEM((Apad,), jnp.float32),
            pltpu.VMEM((Apad,), jnp.int32),
            pltpu.VMEM((4, Apad), jnp.float32),
            pltpu.VMEM((Apad,), jnp.int32),
        ],
    )
    def run(anc_hbm, tgt_hbm, loc_hbm, conf_hbm,
            anc_v, tgt_v, pnt_v, bov_v, bix_v, loc_v, conf_v):
        b = lax.axis_index("s") * 2 + lax.axis_index("c")

        pltpu.sync_copy(anc_hbm, anc_v)
        pltpu.sync_copy(tgt_hbm.at[b], tgt_v)

        lane = lax.iota(jnp.int32, L)

        tx1r = tgt_v[0, :]
        ty1r = tgt_v[1, :]
        tx2r = tgt_v[2, :]
        ty2r = tgt_v[3, :]
        GRP = 8
        UNROLL = 2
        av = jnp.zeros((L,), jnp.int32)
        for g in range(NOBJ // GRP):
            js = list(range(g * GRP, (g + 1) * GRP))
            spl = []
            for j in js:
                x1s = tx1r[j]
                y1s = ty1r[j]
                x2s = tx2r[j]
                y2s = ty2r[j]
                spl.append((
                    jnp.full((L,), x1s, jnp.float32),
                    jnp.full((L,), y1s, jnp.float32),
                    jnp.full((L,), x2s, jnp.float32),
                    jnp.full((L,), y2s, jnp.float32),
                    jnp.full((L,), (x2s - x1s) * (y2s - y1s), jnp.float32),
                ))

            def p1(cc, carry, spl=spl, g=g, js=js):
                ms, cis = carry
                ms = list(ms)
                cis = list(cis)
                for u in range(UNROLL):
                    c = cc * UNROLL + u
                    s = pl.ds(c * L, L)
                    if g == 0:
                        acx = anc_v[0, s]
                        acy = anc_v[1, s]
                        aw = anc_v[2, s]
                        ah = anc_v[3, s]
                        ax1 = acx - aw / 2.0
                        ay1 = acy - ah / 2.0
                        ax2 = acx + aw / 2.0
                        ay2 = acy + ah / 2.0
                        ab = (ax2 - ax1) * (ay2 - ay1)
                        pnt_v[0, s] = ax1
                        pnt_v[1, s] = ay1
                        pnt_v[2, s] = ax2
                        pnt_v[3, s] = ay2
                        pnt_v[4, s] = ab
                        bo = jnp.full((L,), -1.0, jnp.float32)
                        bi = jnp.zeros((L,), jnp.int32)
                    else:
                        ax1 = pnt_v[0, s]
                        ay1 = pnt_v[1, s]
                        ax2 = pnt_v[2, s]
                        ay2 = pnt_v[3, s]
                        ab = pnt_v[4, s]
                        bo = bov_v[s]
                        bi = bix_v[s]
                    for k, j in enumerate(js):
                        vx1, vy1, vx2, vy2, va = spl[k]
                        iw = jnp.maximum(
                            jnp.minimum(vx2, ax2) - jnp.maximum(vx1, ax1), 0.0)
                        ih = jnp.maximum(
                            jnp.minimum(vy2, ay2) - jnp.maximum(vy1, ay1), 0.0)
                        inter = iw * ih
                        iou = inter / (va + ab - inter)
                        upd = iou > ms[k]
                        ms[k] = jnp.where(upd, iou, ms[k])
                        cis[k] = jnp.where(upd, c, cis[k])
                        u2 = iou > bo
                        bo = jnp.where(u2, iou, bo)
                        bi = jnp.where(u2, jnp.int32(j), bi)
                    bix_v[s] = bi
                    if g == 0:
                        bov_v[s] = bo
                    else:
                        r0 = jnp.zeros((L,), jnp.int32)
                        gx1 = plsc.load_gather(tgt_v, [r0, bi])
                        gy1 = plsc.load_gather(tgt_v, [r0 + 1, bi])
                        gx2 = plsc.load_gather(tgt_v, [r0 + 2, bi])
                        gy2 = plsc.load_gather(tgt_v, [r0 + 3, bi])
                        glb = plsc.load_gather(tgt_v, [r0 + 4, bi])
                        acx = anc_v[0, s]
                        acy = anc_v[1, s]
                        aw = anc_v[2, s]
                        ah = anc_v[3, s]
                        loc_v[0, s] = ((gx1 + gx2) / 2.0 - acx) / (VAR0 * aw)
                        loc_v[1, s] = ((gy1 + gy2) / 2.0 - acy) / (VAR0 * ah)
                        loc_v[2, s] = _log_f32((gx2 - gx1) / aw) / VAR1
                        loc_v[3, s] = _log_f32((gy2 - gy1) / ah) / VAR1
                        lbl = glb.astype(jnp.int32) + 1
                        conf_v[s] = jnp.where(bo < OVERLAP_THRESHOLD, 0, lbl)
                return tuple(ms), tuple(cis)

            init = (tuple(jnp.full((L,), -1.0, jnp.float32) for _ in js),
                    tuple(jnp.zeros((L,), jnp.int32) for _ in js))
            ms, cis = plsc.parallel_loop(0, NCHUNK // UNROLL, carry=init)(p1)
            for k, j in enumerate(js):
                best = jnp.max(ms[k])
                acand = jnp.where(ms[k] == best, cis[k] * L + lane,
                                  jnp.int32(1 << 20))
                aj = jnp.min(acand)
                av = jnp.where(lane == j, aj, av)
                onelane = lane == 0
                idxv = jnp.full((L,), aj, jnp.int32)
                if g == 0:
                    plsc.store_scatter(bov_v, [idxv],
                                       jnp.full((L,), 1.0, jnp.float32),
                                       mask=onelane)
                plsc.store_scatter(bix_v, [idxv],
                                   jnp.full((L,), jnp.int32(j)),
                                   mask=onelane)

        r0 = jnp.zeros((L,), jnp.int32)
        bi_f = plsc.load_gather(bix_v, [av])
        gx1 = plsc.load_gather(tgt_v, [r0, bi_f])
        gy1 = plsc.load_gather(tgt_v, [r0 + 1, bi_f])
        gx2 = plsc.load_gather(tgt_v, [r0 + 2, bi_f])
        gy2 = plsc.load_gather(tgt_v, [r0 + 3, bi_f])
        glb = plsc.load_gather(tgt_v, [r0 + 4, bi_f])
        acx = plsc.load_gather(anc_v, [r0, av])
        acy = plsc.load_gather(anc_v, [r0 + 1, av])
        aw = plsc.load_gather(anc_v, [r0 + 2, av])
        ah = plsc.load_gather(anc_v, [r0 + 3, av])
        plsc.store_scatter(loc_v, [r0, av],
                           ((gx1 + gx2) / 2.0 - acx) / (VAR0 * aw))
        plsc.store_scatter(loc_v, [r0 + 1, av],
                           ((gy1 + gy2) / 2.0 - acy) / (VAR0 * ah))
        plsc.store_scatter(loc_v, [r0 + 2, av],
                           _log_f32((gx2 - gx1) / aw) / VAR1)
        plsc.store_scatter(loc_v, [r0 + 3, av],
                           _log_f32((gy2 - gy1) / ah) / VAR1)
        plsc.store_scatter(conf_v, [av], glb.astype(jnp.int32) + 1)

        pltpu.sync_copy(loc_v, loc_hbm.at[b])
        pltpu.sync_copy(conf_v, conf_hbm.at[b])

    return run(anc, tgt)


def kernel(loc_data, conf_data, targets, anchor_boxes):
    del loc_data, conf_data
    B = targets.shape[0]
    A = anchor_boxes.shape[0]
    Apad = ((A + 4 * L - 1) // (4 * L)) * (4 * L)
    npad = Apad - A
    pad_boxes = jnp.concatenate(
        [jnp.full((npad, 2), 4.0, jnp.float32),
         jnp.full((npad, 2), 0.125, jnp.float32)], axis=1)
    anc = jnp.concatenate([anchor_boxes, pad_boxes], axis=0).T
    tgt = jnp.transpose(targets, (0, 2, 1))
    loc_o, conf_o = _sc_match(anc, tgt, B, Apad)
    loc_t = jnp.transpose(loc_o, (0, 2, 1))[:, :A, :]
    conf_t = conf_o[:, :A]
    return loc_t, conf_t


# speedup vs baseline: 1.1794x; 1.1794x over previous
import functools

import jax
import jax.numpy as jnp
from jax import lax
from jax.experimental import pallas as pl
from jax.experimental.pallas import tpu as pltpu
from jax.experimental.pallas import tpu_sc as plsc

L = 16
NOBJ = 16
OVERLAP_THRESHOLD = 0.5
VAR0 = 0.1
VAR1 = 0.2
LN2 = 0.6931471805599453
SQRT2 = 1.4142135623730951


def _log_f32(x):
    xi = lax.bitcast_convert_type(x, jnp.int32)
    e = (xi >> 23) - 127
    mi = (xi & 0x007FFFFF) | 0x3F800000
    m = lax.bitcast_convert_type(mi, jnp.float32)
    big = m > SQRT2
    m = jnp.where(big, m * 0.5, m)
    e = e + jnp.where(big, 1, 0)
    s = (m - 1.0) / (m + 1.0)
    s2 = s * s
    p = 1.0 / 9.0 + s2 * 0.0
    p = 1.0 / 7.0 + s2 * p
    p = 1.0 / 5.0 + s2 * p
    p = 1.0 / 3.0 + s2 * p
    p = 1.0 + s2 * p
    return e.astype(jnp.float32) * LN2 + 2.0 * s * p


def _sc_match(anc, tgt, B, Apad):
    NCHUNK = Apad // L
    mesh = plsc.VectorSubcoreMesh(core_axis_name="c", subcore_axis_name="s")

    @functools.partial(
        pl.kernel,
        out_type=(
            jax.ShapeDtypeStruct((B, 4, Apad), jnp.float32),
            jax.ShapeDtypeStruct((B, Apad), jnp.int32),
        ),
        mesh=mesh,
        compiler_params=pltpu.CompilerParams(needs_layout_passes=False),
        scratch_types=[
            pltpu.VMEM((4, Apad), jnp.float32),
            pltpu.VMEM((5, NOBJ), jnp.float32),
            pltpu.VMEM((5, Apad), jnp.float32),
            pltpu.VMEM((Apad,), jnp.float32),
            pltpu.VMEM((Apad,), jnp.int32),
            pltpu.VMEM((4, Apad), jnp.float32),
            pltpu.VMEM((Apad,), jnp.int32),
        ],
    )
    def run(anc_hbm, tgt_hbm, loc_hbm, conf_hbm,
            anc_v, tgt_v, pnt_v, bov_v, bix_v, loc_v, conf_v):
        b = (lax.axis_index("s") * 2 + lax.axis_index("c")) % B

        pltpu.sync_copy(anc_hbm, anc_v)
        pltpu.sync_copy(tgt_hbm.at[b], tgt_v)

        lane = lax.iota(jnp.int32, L)

        tx1r = tgt_v[0, :]
        ty1r = tgt_v[1, :]
        tx2r = tgt_v[2, :]
        ty2r = tgt_v[3, :]
        GRP = 8
        UNROLL = 2
        av = jnp.zeros((L,), jnp.int32)
        for g in range(NOBJ // GRP):
            js = list(range(g * GRP, (g + 1) * GRP))
            spl = []
            for j in js:
                x1s = tx1r[j]
                y1s = ty1r[j]
                x2s = tx2r[j]
                y2s = ty2r[j]
                spl.append((
                    jnp.full((L,), x1s, jnp.float32),
                    jnp.full((L,), y1s, jnp.float32),
                    jnp.full((L,), x2s, jnp.float32),
                    jnp.full((L,), y2s, jnp.float32),
                    jnp.full((L,), (x2s - x1s) * (y2s - y1s), jnp.float32),
                ))

            def p1(cc, carry, spl=spl, g=g, js=js):
                ms, cis = carry
                ms = list(ms)
                cis = list(cis)
                for u in range(UNROLL):
                    c = cc * UNROLL + u
                    s = pl.ds(c * L, L)
                    if g == 0:
                        acx = anc_v[0, s]
                        acy = anc_v[1, s]
                        aw = anc_v[2, s]
                        ah = anc_v[3, s]
                        ax1 = acx - aw / 2.0
                        ay1 = acy - ah / 2.0
                        ax2 = acx + aw / 2.0
                        ay2 = acy + ah / 2.0
                        ab = (ax2 - ax1) * (ay2 - ay1)
                        pnt_v[0, s] = ax1
                        pnt_v[1, s] = ay1
                        pnt_v[2, s] = ax2
                        pnt_v[3, s] = ay2
                        pnt_v[4, s] = ab
                        bo = jnp.full((L,), -1.0, jnp.float32)
                        bi = jnp.zeros((L,), jnp.int32)
                    else:
                        ax1 = pnt_v[0, s]
                        ay1 = pnt_v[1, s]
                        ax2 = pnt_v[2, s]
                        ay2 = pnt_v[3, s]
                        ab = pnt_v[4, s]
                        bo = bov_v[s]
                        bi = bix_v[s]
                    for k, j in enumerate(js):
                        vx1, vy1, vx2, vy2, va = spl[k]
                        iw = jnp.maximum(
                            jnp.minimum(vx2, ax2) - jnp.maximum(vx1, ax1), 0.0)
                        ih = jnp.maximum(
                            jnp.minimum(vy2, ay2) - jnp.maximum(vy1, ay1), 0.0)
                        inter = iw * ih
                        iou = inter / (va + ab - inter)
                        upd = iou > ms[k]
                        ms[k] = jnp.where(upd, iou, ms[k])
                        cis[k] = jnp.where(upd, c, cis[k])
                        u2 = iou > bo
                        bo = jnp.where(u2, iou, bo)
                        bi = jnp.where(u2, jnp.int32(j), bi)
                    bix_v[s] = bi
                    if g == 0:
                        bov_v[s] = bo
                    else:
                        r0 = jnp.zeros((L,), jnp.int32)
                        gx1 = plsc.load_gather(tgt_v, [r0, bi])
                        gy1 = plsc.load_gather(tgt_v, [r0 + 1, bi])
                        gx2 = plsc.load_gather(tgt_v, [r0 + 2, bi])
                        gy2 = plsc.load_gather(tgt_v, [r0 + 3, bi])
                        glb = plsc.load_gather(tgt_v, [r0 + 4, bi])
                        acx = anc_v[0, s]
                        acy = anc_v[1, s]
                        aw = anc_v[2, s]
                        ah = anc_v[3, s]
                        loc_v[0, s] = ((gx1 + gx2) / 2.0 - acx) / (VAR0 * aw)
                        loc_v[1, s] = ((gy1 + gy2) / 2.0 - acy) / (VAR0 * ah)
                        loc_v[2, s] = _log_f32((gx2 - gx1) / aw) / VAR1
                        loc_v[3, s] = _log_f32((gy2 - gy1) / ah) / VAR1
                        lbl = glb.astype(jnp.int32) + 1
                        conf_v[s] = jnp.where(bo < OVERLAP_THRESHOLD, 0, lbl)
                return tuple(ms), tuple(cis)

            init = (tuple(jnp.full((L,), -1.0, jnp.float32) for _ in js),
                    tuple(jnp.zeros((L,), jnp.int32) for _ in js))
            ms, cis = lax.fori_loop(0, NCHUNK // UNROLL, p1, init)
            for k, j in enumerate(js):
                best = jnp.max(ms[k])
                acand = jnp.where(ms[k] == best, cis[k] * L + lane,
                                  jnp.int32(1 << 20))
                aj = jnp.min(acand)
                av = jnp.where(lane == j, aj, av)
                onelane = lane == 0
                idxv = jnp.full((L,), aj, jnp.int32)
                if g == 0:
                    plsc.store_scatter(bov_v, [idxv],
                                       jnp.full((L,), 1.0, jnp.float32),
                                       mask=onelane)
                plsc.store_scatter(bix_v, [idxv],
                                   jnp.full((L,), jnp.int32(j)),
                                   mask=onelane)

        r0 = jnp.zeros((L,), jnp.int32)
        bi_f = plsc.load_gather(bix_v, [av])
        gx1 = plsc.load_gather(tgt_v, [r0, bi_f])
        gy1 = plsc.load_gather(tgt_v, [r0 + 1, bi_f])
        gx2 = plsc.load_gather(tgt_v, [r0 + 2, bi_f])
        gy2 = plsc.load_gather(tgt_v, [r0 + 3, bi_f])
        glb = plsc.load_gather(tgt_v, [r0 + 4, bi_f])
        acx = plsc.load_gather(anc_v, [r0, av])
        acy = plsc.load_gather(anc_v, [r0 + 1, av])
        aw = plsc.load_gather(anc_v, [r0 + 2, av])
        ah = plsc.load_gather(anc_v, [r0 + 3, av])
        plsc.store_scatter(loc_v, [r0, av],
                           ((gx1 + gx2) / 2.0 - acx) / (VAR0 * aw))
        plsc.store_scatter(loc_v, [r0 + 1, av],
                           ((gy1 + gy2) / 2.0 - acy) / (VAR0 * ah))
        plsc.store_scatter(loc_v, [r0 + 2, av],
                           _log_f32((gx2 - gx1) / aw) / VAR1)
        plsc.store_scatter(loc_v, [r0 + 3, av],
                           _log_f32((gy2 - gy1) / ah) / VAR1)
        plsc.store_scatter(conf_v, [av], glb.astype(jnp.int32) + 1)

        pltpu.sync_copy(loc_v, loc_hbm.at[b])
        pltpu.sync_copy(conf_v, conf_hbm.at[b])

    return run(anc, tgt)


def _tc_match(anc, tgt_tc, Apad):
    Btc = tgt_tc.shape[0]

    def body(anc_ref, tgt_ref, loc_ref, conf_ref):
        acx = anc_ref[0:1, :]
        acy = anc_ref[1:2, :]
        aw = anc_ref[2:3, :]
        ah = anc_ref[3:4, :]
        ax1 = acx - aw / 2.0
        ay1 = acy - ah / 2.0
        ax2 = acx + aw / 2.0
        ay2 = acy + ah / 2.0
        ab = (ax2 - ax1) * (ay2 - ay1)
        tgtb = tgt_ref[0]
        tx1 = tgtb[:, 0:1]
        ty1 = tgtb[:, 1:2]
        tx2 = tgtb[:, 2:3]
        ty2 = tgtb[:, 3:4]
        va = (tx2 - tx1) * (ty2 - ty1)
        iw = jnp.maximum(jnp.minimum(tx2, ax2) - jnp.maximum(tx1, ax1), 0.0)
        ih = jnp.maximum(jnp.minimum(ty2, ay2) - jnp.maximum(ty1, ay1), 0.0)
        inter = iw * ih
        iou = inter / (va + ab - inter)
        bo = iou[0:1, :]
        bi = jnp.zeros((1, Apad), jnp.int32)
        for j in range(1, NOBJ):
            u = iou[j:j + 1, :] > bo
            bo = jnp.where(u, iou[j:j + 1, :], bo)
            bi = jnp.where(u, jnp.int32(j), bi)
        M = jnp.max(iou, axis=1, keepdims=True)
        lin = lax.broadcasted_iota(jnp.int32, (NOBJ, Apad), 1)
        cand = jnp.where(iou == M, lin, jnp.int32(1 << 20))
        ajs = jnp.min(cand, axis=1, keepdims=True)
        lin1 = lin[0:1, :]
        for j in range(NOBJ):
            fm = lin1 == ajs[j, 0]
            bo = jnp.where(fm, 1.0, bo)
            bi = jnp.where(fm, jnp.int32(j), bi)
        gx1 = jnp.broadcast_to(tx1[0, 0], (1, Apad))
        gy1 = jnp.broadcast_to(ty1[0, 0], (1, Apad))
        gx2 = jnp.broadcast_to(tx2[0, 0], (1, Apad))
        gy2 = jnp.broadcast_to(ty2[0, 0], (1, Apad))
        glb = jnp.broadcast_to(tgtb[0, 4], (1, Apad))
        for j in range(1, NOBJ):
            m = bi == j
            gx1 = jnp.where(m, tx1[j, 0], gx1)
            gy1 = jnp.where(m, ty1[j, 0], gy1)
            gx2 = jnp.where(m, tx2[j, 0], gx2)
            gy2 = jnp.where(m, ty2[j, 0], gy2)
            glb = jnp.where(m, tgtb[j, 4], glb)
        loc_ref[0, 0:1, :] = ((gx1 + gx2) / 2.0 - acx) / (VAR0 * aw)
        loc_ref[0, 1:2, :] = ((gy1 + gy2) / 2.0 - acy) / (VAR0 * ah)
        loc_ref[0, 2:3, :] = jnp.log((gx2 - gx1) / aw) / VAR1
        loc_ref[0, 3:4, :] = jnp.log((gy2 - gy1) / ah) / VAR1
        lbl = glb.astype(jnp.int32) + 1
        conf_ref[0, 0:1, :] = jnp.where(bo < OVERLAP_THRESHOLD, 0, lbl)

    loc_o, conf_o = pl.pallas_call(
        body,
        grid=(Btc,),
        in_specs=[
            pl.BlockSpec((4, Apad), lambda b: (0, 0)),
            pl.BlockSpec((1, NOBJ, 5), lambda b: (b, 0, 0)),
        ],
        out_specs=[
            pl.BlockSpec((1, 4, Apad), lambda b: (b, 0, 0)),
            pl.BlockSpec((1, 1, Apad), lambda b: (b, 0, 0)),
        ],
        out_shape=[
            jax.ShapeDtypeStruct((Btc, 4, Apad), jnp.float32),
            jax.ShapeDtypeStruct((Btc, 1, Apad), jnp.int32),
        ],
    )(anc, tgt_tc)
    return loc_o, conf_o.reshape(Btc, Apad)


def kernel(loc_data, conf_data, targets, anchor_boxes):
    del loc_data, conf_data
    B = targets.shape[0]
    A = anchor_boxes.shape[0]
    Apad = ((A + 4 * L - 1) // (4 * L)) * (4 * L)
    npad = Apad - A
    pad_boxes = jnp.concatenate(
        [jnp.full((npad, 2), 4.0, jnp.float32),
         jnp.full((npad, 2), 0.125, jnp.float32)], axis=1)
    anc = jnp.concatenate([anchor_boxes, pad_boxes], axis=0).T
    NSC = 16
    tgt = jnp.transpose(targets[:NSC], (0, 2, 1))
    loc_sc, conf_sc = _sc_match(anc, tgt, NSC, Apad)
    loc_tc, conf_tc = _tc_match(anc, targets[NSC:], Apad)
    loc_o = jnp.concatenate([loc_sc, loc_tc], axis=0)
    conf_o = jnp.concatenate([conf_sc, conf_tc], axis=0)
    loc_t = jnp.transpose(loc_o, (0, 2, 1))[:, :A, :]
    conf_t = conf_o[:, :A]
    return loc_t, conf_t


# speedup vs baseline: 1.2567x; 1.0655x over previous
import functools

import jax
import jax.numpy as jnp
from jax import lax
from jax.experimental import pallas as pl
from jax.experimental.pallas import tpu as pltpu
from jax.experimental.pallas import tpu_sc as plsc

L = 16
NOBJ = 16
OVERLAP_THRESHOLD = 0.5
VAR0 = 0.1
VAR1 = 0.2
LN2 = 0.6931471805599453
SQRT2 = 1.4142135623730951


def _log_f32(x):
    xi = lax.bitcast_convert_type(x, jnp.int32)
    e = (xi >> 23) - 127
    mi = (xi & 0x007FFFFF) | 0x3F800000
    m = lax.bitcast_convert_type(mi, jnp.float32)
    big = m > SQRT2
    m = jnp.where(big, m * 0.5, m)
    e = e + jnp.where(big, 1, 0)
    s = (m - 1.0) / (m + 1.0)
    s2 = s * s
    p = 1.0 / 9.0 + s2 * 0.0
    p = 1.0 / 7.0 + s2 * p
    p = 1.0 / 5.0 + s2 * p
    p = 1.0 / 3.0 + s2 * p
    p = 1.0 + s2 * p
    return e.astype(jnp.float32) * LN2 + 2.0 * s * p


def _sc_match(anc, tgt, B, Apad):
    NCHUNK = Apad // L
    mesh = plsc.VectorSubcoreMesh(core_axis_name="c", subcore_axis_name="s")

    @functools.partial(
        pl.kernel,
        out_type=(
            jax.ShapeDtypeStruct((B, 4, Apad), jnp.float32),
            jax.ShapeDtypeStruct((B, Apad), jnp.int32),
        ),
        mesh=mesh,
        compiler_params=pltpu.CompilerParams(needs_layout_passes=False),
        scratch_types=[
            pltpu.VMEM((4, Apad), jnp.float32),
            pltpu.VMEM((5, NOBJ), jnp.float32),
            pltpu.VMEM((5, Apad), jnp.float32),
            pltpu.VMEM((Apad,), jnp.float32),
            pltpu.VMEM((Apad,), jnp.int32),
            pltpu.VMEM((4, Apad), jnp.float32),
            pltpu.VMEM((Apad,), jnp.int32),
        ],
    )
    def run(anc_hbm, tgt_hbm, loc_hbm, conf_hbm,
            anc_v, tgt_v, pnt_v, bov_v, bix_v, loc_v, conf_v):
        b = lax.axis_index("s") * 2 + lax.axis_index("c")

        pltpu.sync_copy(anc_hbm, anc_v)
        pltpu.sync_copy(tgt_hbm.at[b], tgt_v)

        lane = lax.iota(jnp.int32, L)

        tx1r = tgt_v[0, :]
        ty1r = tgt_v[1, :]
        tx2r = tgt_v[2, :]
        ty2r = tgt_v[3, :]
        GRP = 8
        UNROLL = 2
        av = jnp.zeros((L,), jnp.int32)
        for g in range(NOBJ // GRP):
            js = list(range(g * GRP, (g + 1) * GRP))
            spl = []
            for j in js:
                x1s = tx1r[j]
                y1s = ty1r[j]
                x2s = tx2r[j]
                y2s = ty2r[j]
                spl.append((
                    jnp.full((L,), x1s, jnp.float32),
                    jnp.full((L,), y1s, jnp.float32),
                    jnp.full((L,), x2s, jnp.float32),
                    jnp.full((L,), y2s, jnp.float32),
                    jnp.full((L,), (x2s - x1s) * (y2s - y1s), jnp.float32),
                ))

            def p1(cc, carry, spl=spl, g=g, js=js):
                ms, cis = carry
                ms = list(ms)
                cis = list(cis)
                for u in range(UNROLL):
                    c = cc * UNROLL + u
                    s = pl.ds(c * L, L)
                    if g == 0:
                        acx = anc_v[0, s]
                        acy = anc_v[1, s]
                        aw = anc_v[2, s]
                        ah = anc_v[3, s]
                        ax1 = acx - aw / 2.0
                        ay1 = acy - ah / 2.0
                        ax2 = acx + aw / 2.0
                        ay2 = acy + ah / 2.0
                        ab = (ax2 - ax1) * (ay2 - ay1)
                        pnt_v[0, s] = ax1
                        pnt_v[1, s] = ay1
                        pnt_v[2, s] = ax2
                        pnt_v[3, s] = ay2
                        pnt_v[4, s] = ab
                        bo = jnp.full((L,), -1.0, jnp.float32)
                        bi = jnp.zeros((L,), jnp.int32)
                    else:
                        ax1 = pnt_v[0, s]
                        ay1 = pnt_v[1, s]
                        ax2 = pnt_v[2, s]
                        ay2 = pnt_v[3, s]
                        ab = pnt_v[4, s]
                        bo = bov_v[s]
                        bi = bix_v[s]
                    for k, j in enumerate(js):
                        vx1, vy1, vx2, vy2, va = spl[k]
                        iw = jnp.maximum(
                            jnp.minimum(vx2, ax2) - jnp.maximum(vx1, ax1), 0.0)
                        ih = jnp.maximum(
                            jnp.minimum(vy2, ay2) - jnp.maximum(vy1, ay1), 0.0)
                        inter = iw * ih
                        iou = inter / (va + ab - inter)
                        upd = iou > ms[k]
                        ms[k] = jnp.where(upd, iou, ms[k])
                        cis[k] = jnp.where(upd, c, cis[k])
                        u2 = iou > bo
                        bo = jnp.where(u2, iou, bo)
                        bi = jnp.where(u2, jnp.int32(j), bi)
                    bix_v[s] = bi
                    if g == 0:
                        bov_v[s] = bo
                    else:
                        r0 = jnp.zeros((L,), jnp.int32)
                        gx1 = plsc.load_gather(tgt_v, [r0, bi])
                        gy1 = plsc.load_gather(tgt_v, [r0 + 1, bi])
                        gx2 = plsc.load_gather(tgt_v, [r0 + 2, bi])
                        gy2 = plsc.load_gather(tgt_v, [r0 + 3, bi])
                        glb = plsc.load_gather(tgt_v, [r0 + 4, bi])
                        acx = anc_v[0, s]
                        acy = anc_v[1, s]
                        aw = anc_v[2, s]
                        ah = anc_v[3, s]
                        loc_v[0, s] = ((gx1 + gx2) / 2.0 - acx) / (VAR0 * aw)
                        loc_v[1, s] = ((gy1 + gy2) / 2.0 - acy) / (VAR0 * ah)
                        loc_v[2, s] = _log_f32((gx2 - gx1) / aw) / VAR1
                        loc_v[3, s] = _log_f32((gy2 - gy1) / ah) / VAR1
                        lbl = glb.astype(jnp.int32) + 1
                        conf_v[s] = jnp.where(bo < OVERLAP_THRESHOLD, 0, lbl)
                return tuple(ms), tuple(cis)

            init = (tuple(jnp.full((L,), -1.0, jnp.float32) for _ in js),
                    tuple(jnp.zeros((L,), jnp.int32) for _ in js))
            ms, cis = lax.fori_loop(0, NCHUNK // UNROLL, p1, init)
            for k, j in enumerate(js):
                best = jnp.max(ms[k])
                acand = jnp.where(ms[k] == best, cis[k] * L + lane,
                                  jnp.int32(1 << 20))
                aj = jnp.min(acand)
                av = jnp.where(lane == j, aj, av)
                onelane = lane == 0
                idxv = jnp.full((L,), aj, jnp.int32)
                if g == 0:
                    plsc.store_scatter(bov_v, [idxv],
                                       jnp.full((L,), 1.0, jnp.float32),
                                       mask=onelane)
                plsc.store_scatter(bix_v, [idxv],
                                   jnp.full((L,), jnp.int32(j)),
                                   mask=onelane)

        r0 = jnp.zeros((L,), jnp.int32)
        bi_f = plsc.load_gather(bix_v, [av])
        gx1 = plsc.load_gather(tgt_v, [r0, bi_f])
        gy1 = plsc.load_gather(tgt_v, [r0 + 1, bi_f])
        gx2 = plsc.load_gather(tgt_v, [r0 + 2, bi_f])
        gy2 = plsc.load_gather(tgt_v, [r0 + 3, bi_f])
        glb = plsc.load_gather(tgt_v, [r0 + 4, bi_f])
        acx = plsc.load_gather(anc_v, [r0, av])
        acy = plsc.load_gather(anc_v, [r0 + 1, av])
        aw = plsc.load_gather(anc_v, [r0 + 2, av])
        ah = plsc.load_gather(anc_v, [r0 + 3, av])
        plsc.store_scatter(loc_v, [r0, av],
                           ((gx1 + gx2) / 2.0 - acx) / (VAR0 * aw))
        plsc.store_scatter(loc_v, [r0 + 1, av],
                           ((gy1 + gy2) / 2.0 - acy) / (VAR0 * ah))
        plsc.store_scatter(loc_v, [r0 + 2, av],
                           _log_f32((gx2 - gx1) / aw) / VAR1)
        plsc.store_scatter(loc_v, [r0 + 3, av],
                           _log_f32((gy2 - gy1) / ah) / VAR1)
        plsc.store_scatter(conf_v, [av], glb.astype(jnp.int32) + 1)

        pltpu.sync_copy(loc_v, loc_hbm.at[b])
        pltpu.sync_copy(conf_v, conf_hbm.at[b])

    return run(anc, tgt)


def kernel(loc_data, conf_data, targets, anchor_boxes):
    del loc_data, conf_data
    B = targets.shape[0]
    A = anchor_boxes.shape[0]
    Apad = ((A + 4 * L - 1) // (4 * L)) * (4 * L)
    npad = Apad - A
    pad_boxes = jnp.concatenate(
        [jnp.full((npad, 2), 4.0, jnp.float32),
         jnp.full((npad, 2), 0.125, jnp.float32)], axis=1)
    anc = jnp.concatenate([anchor_boxes, pad_boxes], axis=0).T
    tgt = jnp.transpose(targets, (0, 2, 1))
    loc_o, conf_o = _sc_match(anc, tgt, B, Apad)
    loc_t = jnp.transpose(loc_o, (0, 2, 1))[:, :A, :]
    conf_t = conf_o[:, :A]
    return loc_t, conf_t
